# baseline (speedup 1.0000x reference)
import math

import jax
import jax.numpy as jnp
from jax.experimental import pallas as pl
from jax.experimental.pallas import tpu as pltpu

_GROUP = 4
_BUDGET = 0.5
_LANE = 128


def _body(sal_ref, msk_ref, wt_ref, b_ref, out_ref, lasso_ref,
          acc_ref, mask_acc_ref, *, n_s, s_blk, n_b, n_ch, k_drop):
    bi = pl.program_id(0)
    si = pl.program_id(1)

    sal = sal_ref[0]
    m = msk_ref[0]
    prod = sal * m

    n_fold = s_blk // _LANE
    part = prod[:, 0:_LANE]
    mpart = m[:, 0:_LANE]
    for f in range(1, n_fold):
        part = part + prod[:, f * _LANE:(f + 1) * _LANE]
        mpart = mpart + m[:, f * _LANE:(f + 1) * _LANE]

    @pl.when(si == 0)
    def _init():
        acc_ref[bi] = part
        mask_acc_ref[pl.ds(bi, 1), :] = mpart

    @pl.when(si != 0)
    def _accum():
        acc_ref[bi] = acc_ref[bi] + part
        mask_acc_ref[pl.ds(bi, 1), :] = mask_acc_ref[pl.ds(bi, 1), :] + mpart

    @pl.when(jnp.logical_and(bi == n_b - 1, si == n_s - 1))
    def _finalize():
        total = float(n_s * s_blk)
        pooled = jnp.sum(acc_ref[:], axis=2) / total
        active = jnp.sum(mask_acc_ref[:], axis=1, keepdims=True) + 0.0001
        pooled = pooled * total / active
        z = jax.nn.sigmoid(
            jnp.dot(pooled, wt_ref[:], preferred_element_type=jnp.float32)
            + b_ref[:])
        lasso_ref[:] = jnp.full((1, 1), jnp.mean(jnp.sum(z, axis=-1)),
                                jnp.float32)

        zi = z[:, :, None]
        zj = z[:, None, :]
        ii = jax.lax.broadcasted_iota(jnp.int32, (n_b, n_ch, n_ch), 1)
        jj = jax.lax.broadcasted_iota(jnp.int32, (n_b, n_ch, n_ch), 2)
        below = jnp.logical_or(zj < zi,
                               jnp.logical_and(zj == zi, jj < ii))
        cnt = jnp.sum(below.astype(jnp.int32), axis=2)
        keep = jnp.logical_and(cnt >= k_drop, z > 0)

        n_out = n_ch * _GROUP
        row = jax.lax.broadcasted_iota(jnp.int32, (n_ch, n_out), 0)
        col = jax.lax.broadcasted_iota(jnp.int32, (n_ch, n_out), 1)
        expand = (row == col // _GROUP).astype(jnp.float32)
        out_ref[:] = jnp.dot(keep.astype(jnp.float32), expand,
                             preferred_element_type=jnp.float32
                             ).astype(jnp.int32)


def kernel(x, saliency_mask, mask_hard, W, b):
    B, C, H, Wd = saliency_mask.shape
    S = H * Wd
    F = W.shape[0]
    k_drop = math.ceil((1.0 - _BUDGET) * F)

    s_blk = 3584
    n_s = S // s_blk

    sal = saliency_mask.reshape(B, C, S)
    msk = mask_hard.reshape(B, 1, S)
    wt = W.T
    b2 = b.reshape(1, F)

    expanded, lasso = pl.pallas_call(
        lambda *refs: _body(*refs, n_s=n_s, s_blk=s_blk, n_b=B, n_ch=F,
                            k_drop=k_drop),
        grid=(B, n_s),
        in_specs=[
            pl.BlockSpec((1, C, s_blk), lambda bi, si: (bi, 0, si)),
            pl.BlockSpec((1, 1, s_blk), lambda bi, si: (bi, 0, si)),
            pl.BlockSpec((C, F), lambda bi, si: (0, 0)),
            pl.BlockSpec((1, F), lambda bi, si: (0, 0)),
        ],
        out_specs=[
            pl.BlockSpec((B, F * _GROUP), lambda bi, si: (0, 0)),
            pl.BlockSpec((1, 1), lambda bi, si: (0, 0)),
        ],
        out_shape=[
            jax.ShapeDtypeStruct((B, F * _GROUP), jnp.int32),
            jax.ShapeDtypeStruct((1, 1), jnp.float32),
        ],
        scratch_shapes=[
            pltpu.VMEM((B, C, _LANE), jnp.float32),
            pltpu.VMEM((B, _LANE), jnp.float32),
        ],
    )(sal, msk, wt, b2)

    return expanded, lasso.reshape(())
